# baseline (speedup 1.0000x reference)
import functools

import jax
import jax.numpy as jnp
from jax import lax
from jax.experimental import pallas as pl
from jax.experimental.pallas import tpu as pltpu
from jax.experimental.pallas import tpu_sc as plsc

CH = 128
KB = 8
BT = 1024


def _sc_gather_body(NC, WV_CH, CV_CH,
                    wv_idx, cv_idx, wv_tab, cv_tab,
                    wv_out, cv_out, idx_wv, idx_cv, buf, sem):
    cid = lax.axis_index("c")
    sid = lax.axis_index("s")
    wid = sid * NC + cid

    pltpu.sync_copy(wv_idx.at[pl.ds(wid * WV_CH, WV_CH)], idx_wv)
    pltpu.sync_copy(cv_idx.at[pl.ds(wid * CV_CH, CV_CH)], idx_cv)

    def run(tab, idx_v, out, n_outer, rows_pw):
        def body(j, carry):
            cps = [
                pltpu.async_copy(tab.at[idx_v.at[j * KB + b]],
                                 buf.at[pl.ds(b * CH, CH)], sem)
                for b in range(KB)
            ]
            for c in cps:
                c.wait()
            pltpu.sync_copy(
                buf, out.at[pl.ds(wid * rows_pw + j * (KB * CH), KB * CH)])
            return carry
        lax.fori_loop(0, n_outer, body, 0)

    run(wv_tab, idx_wv, wv_out, WV_CH // KB, WV_CH * CH)
    run(cv_tab, idx_cv, cv_out, CV_CH // KB, CV_CH * CH)


def _tc_loss_body(B, *refs):
    wv_refs = refs[0:6]
    cv_refs = refs[6:16]
    out_ref = refs[16]

    def renorm(x):
        sos = jnp.sum(x * x, axis=1, keepdims=True)
        n = jnp.sqrt(sos)
        return x * jnp.minimum(1.0, 1.0 / jnp.maximum(n, 1e-7))

    cont = renorm(cv_refs[0][:])
    for r in cv_refs[1:]:
        cont = cont + renorm(r[:])
    cont = cont * 0.1

    cent = renorm(wv_refs[0][:])
    pos = jnp.sum(cont * cent, axis=1)
    acc = jnp.sum(jnp.log1p(jnp.exp(-pos))) * (1.0 / B)
    for k in range(1, 6):
        dk = jnp.sum(cont * renorm(wv_refs[k][:]), axis=1)
        acc = acc + jnp.sum(jnp.log1p(jnp.exp(dk))) * (1.0 / (5.0 * B))

    prev = jnp.where(pl.program_id(0) == 0, 0.0, out_ref[0, 0])
    out_ref[0, 0] = prev + acc


def kernel(context, center, negcase, wordvec, contextvec):
    B = center.shape[0]
    D = wordvec.shape[1]

    info = plsc.get_sparse_core_info()
    NC, NS = info.num_cores, info.num_subcores
    NW = NC * NS

    cen = center.astype(jnp.int32).reshape(-1)
    neg = negcase.astype(jnp.int32).T.reshape(-1)
    ctx = context.astype(jnp.int32).T.reshape(-1)
    wv_idx = jnp.concatenate([cen, neg]).reshape(-1, CH)
    cv_idx = ctx.reshape(-1, CH)

    WV_CH = (6 * B) // (NW * CH)
    CV_CH = (10 * B) // (NW * CH)

    mesh = plsc.VectorSubcoreMesh(core_axis_name="c", subcore_axis_name="s")
    sc_gather = functools.partial(
        pl.kernel,
        mesh=mesh,
        out_type=[
            jax.ShapeDtypeStruct((6 * B, D), jnp.float32),
            jax.ShapeDtypeStruct((10 * B, D), jnp.float32),
        ],
        scratch_types=[
            pltpu.VMEM((WV_CH, CH), jnp.int32),
            pltpu.VMEM((CV_CH, CH), jnp.int32),
            pltpu.VMEM((KB * CH, D), jnp.float32),
            pltpu.SemaphoreType.DMA,
        ],
        compiler_params=pltpu.CompilerParams(use_tc_tiling_on_sc=False),
    )(functools.partial(_sc_gather_body, NC, WV_CH, CV_CH))

    wv_rows, cv_rows = sc_gather(wv_idx, cv_idx, wordvec, contextvec)

    grid = B // BT
    in_specs = (
        [pl.BlockSpec((BT, D), lambda i, r=r: (r * grid + i, 0))
         for r in range(6)]
        + [pl.BlockSpec((BT, D), lambda i, r=r: (r * grid + i, 0))
           for r in range(10)]
    )
    out = pl.pallas_call(
        functools.partial(_tc_loss_body, B),
        grid=(grid,),
        in_specs=in_specs,
        out_specs=pl.BlockSpec(memory_space=pltpu.SMEM),
        out_shape=jax.ShapeDtypeStruct((1, 1), jnp.float32),
    )(*([wv_rows] * 6 + [cv_rows] * 10))
    return out[0, 0]


# speedup vs baseline: 1.0088x; 1.0088x over previous
import functools

import jax
import jax.numpy as jnp
from jax import lax
from jax.experimental import pallas as pl
from jax.experimental.pallas import tpu as pltpu
from jax.experimental.pallas import tpu_sc as plsc

CH = 128
KB = 4
BT = 1024
DP = 128


def _sc_gather_body(NC, WV_CH, CV_CH,
                    wv_idx, cv_idx, wv_tab, cv_tab,
                    wv_out, cv_out, idx_wv, idx_cv, buf, sem):
    cid = lax.axis_index("c")
    sid = lax.axis_index("s")
    wid = sid * NC + cid

    pltpu.sync_copy(wv_idx.at[pl.ds(wid * WV_CH, WV_CH)], idx_wv)
    pltpu.sync_copy(cv_idx.at[pl.ds(wid * CV_CH, CV_CH)], idx_cv)

    def run(tab, idx_v, out, n_outer, rows_pw):
        def body(j, carry):
            cps = [
                pltpu.async_copy(tab.at[idx_v.at[j * KB + b]],
                                 buf.at[pl.ds(b * CH, CH)], sem)
                for b in range(KB)
            ]
            for c in cps:
                c.wait()
            pltpu.sync_copy(
                buf, out.at[pl.ds(wid * rows_pw + j * (KB * CH), KB * CH)])
            return carry
        lax.fori_loop(0, n_outer, body, 0)

    run(wv_tab, idx_wv, wv_out, WV_CH // KB, WV_CH * CH)
    run(cv_tab, idx_cv, cv_out, CV_CH // KB, CV_CH * CH)


def _tc_loss_body(B, *refs):
    wv_refs = refs[0:6]
    cv_refs = refs[6:16]
    out_ref = refs[16]

    def renorm(x):
        sos = jnp.sum(x * x, axis=1, keepdims=True)
        n = jnp.sqrt(sos)
        return x * jnp.minimum(1.0, 1.0 / jnp.maximum(n, 1e-7))

    cont = renorm(cv_refs[0][:])
    for r in cv_refs[1:]:
        cont = cont + renorm(r[:])
    cont = cont * 0.1

    cent = renorm(wv_refs[0][:])
    pos = jnp.sum(cont * cent, axis=1)
    acc = jnp.sum(jnp.log1p(jnp.exp(-pos))) * (1.0 / B)
    for k in range(1, 6):
        dk = jnp.sum(cont * renorm(wv_refs[k][:]), axis=1)
        acc = acc + jnp.sum(jnp.log1p(jnp.exp(dk))) * (1.0 / (5.0 * B))

    prev = jnp.where(pl.program_id(0) == 0, 0.0, out_ref[0, 0])
    out_ref[0, 0] = prev + acc


def kernel(context, center, negcase, wordvec, contextvec):
    B = center.shape[0]
    D = wordvec.shape[1]

    wv_p = jnp.pad(wordvec, ((0, 0), (0, DP - D)))
    cv_p = jnp.pad(contextvec, ((0, 0), (0, DP - D)))

    info = plsc.get_sparse_core_info()
    NC, NS = info.num_cores, info.num_subcores
    NW = NC * NS

    cen = center.astype(jnp.int32).reshape(-1)
    neg = negcase.astype(jnp.int32).T.reshape(-1)
    ctx = context.astype(jnp.int32).T.reshape(-1)
    wv_idx = jnp.concatenate([cen, neg]).reshape(-1, CH)
    cv_idx = ctx.reshape(-1, CH)

    WV_CH = (6 * B) // (NW * CH)
    CV_CH = (10 * B) // (NW * CH)

    mesh = plsc.VectorSubcoreMesh(core_axis_name="c", subcore_axis_name="s")
    sc_gather = functools.partial(
        pl.kernel,
        mesh=mesh,
        out_type=[
            jax.ShapeDtypeStruct((6 * B, DP), jnp.float32),
            jax.ShapeDtypeStruct((10 * B, DP), jnp.float32),
        ],
        scratch_types=[
            pltpu.VMEM((WV_CH, CH), jnp.int32),
            pltpu.VMEM((CV_CH, CH), jnp.int32),
            pltpu.VMEM((KB * CH, DP), jnp.float32),
            pltpu.SemaphoreType.DMA,
        ],
        compiler_params=pltpu.CompilerParams(use_tc_tiling_on_sc=True),
    )(functools.partial(_sc_gather_body, NC, WV_CH, CV_CH))

    wv_rows, cv_rows = sc_gather(wv_idx, cv_idx, wv_p, cv_p)

    grid = B // BT
    in_specs = (
        [pl.BlockSpec((BT, DP), lambda i, r=r: (r * grid + i, 0))
         for r in range(6)]
        + [pl.BlockSpec((BT, DP), lambda i, r=r: (r * grid + i, 0))
           for r in range(10)]
    )
    out = pl.pallas_call(
        functools.partial(_tc_loss_body, B),
        grid=(grid,),
        in_specs=in_specs,
        out_specs=pl.BlockSpec(memory_space=pltpu.SMEM),
        out_shape=jax.ShapeDtypeStruct((1, 1), jnp.float32),
    )(*([wv_rows] * 6 + [cv_rows] * 10))
    return out[0, 0]


# speedup vs baseline: 1.8016x; 1.7859x over previous
import functools

import jax
import jax.numpy as jnp
from jax import lax
from jax.experimental import pallas as pl
from jax.experimental.pallas import tpu as pltpu
from jax.experimental.pallas import tpu_sc as plsc

CH = 128
KB = 4
BT = 1024
DP = 128


def _sc_gather_body(NC, WV_CH, CV_CH,
                    wv_idx, cv_idx, wv_tab, cv_tab,
                    wv_out, cv_out, idx_wv, idx_cv, buf, sem):
    cid = lax.axis_index("c")
    sid = lax.axis_index("s")
    wid = sid * NC + cid

    pltpu.sync_copy(wv_idx.at[pl.ds(wid * WV_CH, WV_CH)], idx_wv)
    pltpu.sync_copy(cv_idx.at[pl.ds(wid * CV_CH, CV_CH)], idx_cv)

    def run(tab, idx_v, out, n_outer, rows_pw):
        def body(j, carry):
            cps = [
                pltpu.async_copy(tab.at[idx_v.at[j * KB + b]],
                                 buf.at[pl.ds(b * CH, CH)], sem)
                for b in range(KB)
            ]
            for c in cps:
                c.wait()
            pltpu.sync_copy(
                buf, out.at[pl.ds(wid * rows_pw + j * (KB * CH), KB * CH)])
            return carry
        lax.fori_loop(0, n_outer, body, 0)

    run(wv_tab, idx_wv, wv_out, WV_CH // KB, WV_CH * CH)
    run(cv_tab, idx_cv, cv_out, CV_CH // KB, CV_CH * CH)


def _tc_loss_body(B, *refs):
    wv_refs = refs[0:6]
    cv_refs = refs[6:16]
    out_ref = refs[16]

    def renorm(x):
        sos = jnp.sum(x * x, axis=1, keepdims=True)
        n = jnp.sqrt(sos)
        return x * jnp.minimum(1.0, 1.0 / jnp.maximum(n, 1e-7))

    cont = renorm(cv_refs[0][:])
    for r in cv_refs[1:]:
        cont = cont + renorm(r[:])
    cont = cont * 0.1

    cent = renorm(wv_refs[0][:])
    pos = jnp.sum(cont * cent, axis=1)
    acc = jnp.sum(jnp.log1p(jnp.exp(-pos))) * (1.0 / B)
    for k in range(1, 6):
        dk = jnp.sum(cont * renorm(wv_refs[k][:]), axis=1)
        acc = acc + jnp.sum(jnp.log1p(jnp.exp(dk))) * (1.0 / (5.0 * B))

    prev = jnp.where(pl.program_id(0) == 0, 0.0, out_ref[0, 0])
    out_ref[0, 0] = prev + acc


def kernel(context, center, negcase, wordvec, contextvec):
    B = center.shape[0]
    D = wordvec.shape[1]

    eye_p = jnp.eye(D, DP, dtype=jnp.float32)
    wv_p = wordvec @ eye_p
    cv_p = contextvec @ eye_p

    info = plsc.get_sparse_core_info()
    NC, NS = info.num_cores, info.num_subcores
    NW = NC * NS

    cen = center.astype(jnp.int32).reshape(-1)
    neg = negcase.astype(jnp.int32).T.reshape(-1)
    ctx = context.astype(jnp.int32).T.reshape(-1)
    wv_idx = jnp.concatenate([cen, neg]).reshape(-1, CH)
    cv_idx = ctx.reshape(-1, CH)

    WV_CH = (6 * B) // (NW * CH)
    CV_CH = (10 * B) // (NW * CH)

    mesh = plsc.VectorSubcoreMesh(core_axis_name="c", subcore_axis_name="s")
    sc_gather = functools.partial(
        pl.kernel,
        mesh=mesh,
        out_type=[
            jax.ShapeDtypeStruct((6 * B, DP), jnp.float32),
            jax.ShapeDtypeStruct((10 * B, DP), jnp.float32),
        ],
        scratch_types=[
            pltpu.VMEM((WV_CH, CH), jnp.int32),
            pltpu.VMEM((CV_CH, CH), jnp.int32),
            pltpu.VMEM((KB * CH, DP), jnp.float32),
            pltpu.SemaphoreType.DMA,
        ],
        compiler_params=pltpu.CompilerParams(use_tc_tiling_on_sc=True),
    )(functools.partial(_sc_gather_body, NC, WV_CH, CV_CH))

    wv_rows, cv_rows = sc_gather(wv_idx, cv_idx, wv_p, cv_p)

    grid = B // BT
    in_specs = (
        [pl.BlockSpec((BT, DP), lambda i, r=r: (r * grid + i, 0))
         for r in range(6)]
        + [pl.BlockSpec((BT, DP), lambda i, r=r: (r * grid + i, 0))
           for r in range(10)]
    )
    out = pl.pallas_call(
        functools.partial(_tc_loss_body, B),
        grid=(grid,),
        in_specs=in_specs,
        out_specs=pl.BlockSpec(memory_space=pltpu.SMEM),
        out_shape=jax.ShapeDtypeStruct((1, 1), jnp.float32),
    )(*([wv_rows] * 6 + [cv_rows] * 10))
    return out[0, 0]


# speedup vs baseline: 1.8214x; 1.0110x over previous
import functools

import jax
import jax.numpy as jnp
from jax import lax
from jax.experimental import pallas as pl
from jax.experimental.pallas import tpu as pltpu
from jax.experimental.pallas import tpu_sc as plsc

CH = 128
KB = 4
BT = 1024
DP = 128


def _sc_gather_body(NC, N_CH, idx_hbm, tab, out, idx_v, buf, sem):
    cid = lax.axis_index("c")
    sid = lax.axis_index("s")
    wid = sid * NC + cid
    rows_pw = N_CH * CH

    pltpu.sync_copy(idx_hbm.at[pl.ds(wid * N_CH, N_CH)], idx_v)

    def body(j, carry):
        cps = [
            pltpu.async_copy(tab.at[idx_v.at[j * KB + b]],
                             buf.at[pl.ds(b * CH, CH)], sem)
            for b in range(KB)
        ]
        for c in cps:
            c.wait()
        pltpu.sync_copy(
            buf, out.at[pl.ds(wid * rows_pw + j * (KB * CH), KB * CH)])
        return carry
    lax.fori_loop(0, N_CH // KB, body, 0)


def _tc_loss_body(B, *refs):
    wv_refs = refs[0:6]
    cv_refs = refs[6:16]
    out_ref = refs[16]

    def renorm(x):
        sos = jnp.sum(x * x, axis=1, keepdims=True)
        n = jnp.sqrt(sos)
        return x * jnp.minimum(1.0, 1.0 / jnp.maximum(n, 1e-7))

    cont = renorm(cv_refs[0][:])
    for r in cv_refs[1:]:
        cont = cont + renorm(r[:])
    cont = cont * 0.1

    cent = renorm(wv_refs[0][:])
    pos = jnp.sum(cont * cent, axis=1)
    acc = jnp.sum(jnp.log1p(jnp.exp(-pos))) * (1.0 / B)
    for k in range(1, 6):
        dk = jnp.sum(cont * renorm(wv_refs[k][:]), axis=1)
        acc = acc + jnp.sum(jnp.log1p(jnp.exp(dk))) * (1.0 / (5.0 * B))

    prev = jnp.where(pl.program_id(0) == 0, 0.0, out_ref[0, 0])
    out_ref[0, 0] = prev + acc


def kernel(context, center, negcase, wordvec, contextvec):
    B = center.shape[0]
    D = wordvec.shape[1]

    eye_p = jnp.eye(D, DP, dtype=jnp.float32)
    wv_p = wordvec @ eye_p
    cv_p = contextvec @ eye_p

    info = plsc.get_sparse_core_info()
    NC, NS = info.num_cores, info.num_subcores
    NW = NC * NS

    cen = center.astype(jnp.int32).reshape(-1)
    neg = negcase.astype(jnp.int32).T.reshape(-1)
    ctx = context.astype(jnp.int32).T.reshape(-1)
    wv_idx = jnp.concatenate([cen, neg]).reshape(-1, CH)
    cv_idx = ctx.reshape(-1, CH)

    WV_CH = (6 * B) // (NW * CH)
    CV_CH = (10 * B) // (NW * CH)

    mesh = plsc.VectorSubcoreMesh(core_axis_name="c", subcore_axis_name="s")
    def make_gather(n_rows, n_ch):
        return functools.partial(
            pl.kernel,
            mesh=mesh,
            out_type=jax.ShapeDtypeStruct((n_rows, DP), jnp.float32),
            scratch_types=[
                pltpu.VMEM((n_ch, CH), jnp.int32),
                pltpu.VMEM((KB * CH, DP), jnp.float32),
                pltpu.SemaphoreType.DMA,
            ],
            compiler_params=pltpu.CompilerParams(use_tc_tiling_on_sc=True),
        )(functools.partial(_sc_gather_body, NC, n_ch))

    wv_rows = make_gather(6 * B, WV_CH)(wv_idx, wv_p)
    cv_rows = make_gather(10 * B, CV_CH)(cv_idx, cv_p)

    grid = B // BT
    in_specs = (
        [pl.BlockSpec((BT, DP), lambda i, r=r: (r * grid + i, 0))
         for r in range(6)]
        + [pl.BlockSpec((BT, DP), lambda i, r=r: (r * grid + i, 0))
           for r in range(10)]
    )
    out = pl.pallas_call(
        functools.partial(_tc_loss_body, B),
        grid=(grid,),
        in_specs=in_specs,
        out_specs=pl.BlockSpec(memory_space=pltpu.SMEM),
        out_shape=jax.ShapeDtypeStruct((1, 1), jnp.float32),
    )(*([wv_rows] * 6 + [cv_rows] * 10))
    return out[0, 0]
